# Initial kernel scaffold; baseline (speedup 1.0000x reference)
import jax
import jax.numpy as jnp
from jax.experimental import pallas as pl


def kernel(boxes, scores, class_ids):
    raise NotImplementedError("write your pallas kernel here")


# speedup vs baseline: 29.7062x; 29.7062x over previous
import jax
import jax.numpy as jnp
from jax.experimental import pallas as pl
from jax.experimental.pallas import tpu as pltpu

_N = 20000
_NPAD = 20480
_LANES = 128
_ROWS = _NPAD // _LANES
_BLK_ROWS = 8
_NBLK = _ROWS // _BLK_ROWS
_IOU_T = 0.5


def _nms_body(x1_ref, y1_ref, x2_ref, y2_ref, keep_ref, area_ref):
    area_ref[...] = (x2_ref[...] - x1_ref[...]) * (y2_ref[...] - y1_ref[...])
    keep_ref[...] = jnp.ones((_ROWS, _LANES), jnp.float32)

    for b in range(_NBLK):
        r0 = b * _BLK_ROWS
        t_rows = _ROWS - r0
        row_iota = jax.lax.broadcasted_iota(jnp.int32, (t_rows, _LANES), 0)
        col_iota = jax.lax.broadcasted_iota(jnp.int32, (t_rows, _LANES), 1)
        gidx_t = row_iota * _LANES + col_iota + r0 * _LANES

        def body(i, carry, r0=r0, t_rows=t_rows, gidx_t=gidx_t):
            r = r0 + i // _LANES
            c = i % _LANES
            gi = r0 * _LANES + i

            def pick(ref):
                row = ref[pl.ds(r, 1), :]
                return pltpu.roll(row, -c, 1)[0:1, 0:1]

            x1i = pick(x1_ref)
            y1i = pick(y1_ref)
            x2i = pick(x2_ref)
            y2i = pick(y2_ref)
            ai = pick(area_ref)
            ki = pick(keep_ref)

            tsl = (pl.ds(r0, t_rows), slice(None))
            xx1 = jnp.maximum(x1i, x1_ref[tsl])
            yy1 = jnp.maximum(y1i, y1_ref[tsl])
            xx2 = jnp.minimum(x2i, x2_ref[tsl])
            yy2 = jnp.minimum(y2i, y2_ref[tsl])
            inter = jnp.maximum(xx2 - xx1, 0.0) * jnp.maximum(yy2 - yy1, 0.0)
            iou = inter / (ai + area_ref[tsl] - inter)
            sup = (iou >= _IOU_T) & (gidx_t > gi) & (ki > 0.0)
            keep_ref[tsl] = jnp.where(sup, 0.0, keep_ref[tsl])
            return carry

        jax.lax.fori_loop(0, _BLK_ROWS * _LANES, body, 0)


def _run_nms(x1, y1, x2, y2):
    return pl.pallas_call(
        _nms_body,
        out_shape=jax.ShapeDtypeStruct((_ROWS, _LANES), jnp.float32),
        scratch_shapes=[pltpu.VMEM((_ROWS, _LANES), jnp.float32)],
    )(x1, y1, x2, y2)


def kernel(boxes, scores, class_ids):
    order = jnp.argsort(-scores)
    max_coordinate = boxes.max()
    offsets = class_ids.astype(boxes.dtype) * (max_coordinate + 1.0)
    bb = (boxes + offsets[:, None])[order]

    pad = _NPAD - _N
    zero_pad = jnp.zeros((pad,), jnp.float32)
    neg_pad = jnp.full((pad,), -1.0, jnp.float32)
    x1 = jnp.concatenate([bb[:, 0], zero_pad]).reshape(_ROWS, _LANES)
    y1 = jnp.concatenate([bb[:, 1], zero_pad]).reshape(_ROWS, _LANES)
    x2 = jnp.concatenate([bb[:, 2], neg_pad]).reshape(_ROWS, _LANES)
    y2 = jnp.concatenate([bb[:, 3], neg_pad]).reshape(_ROWS, _LANES)

    keep_sorted = _run_nms(x1, y1, x2, y2).reshape(-1)[:_N]
    mask = jnp.zeros((_N,), jnp.float32).at[order].set(keep_sorted)
    kept_boxes = boxes * mask[:, None]
    return mask, kept_boxes


# speedup vs baseline: 313.1143x; 10.5404x over previous
import functools

import jax
import jax.numpy as jnp
from jax.experimental import pallas as pl
from jax.experimental.pallas import tpu as pltpu

_N = 20000
_B = 512
_NPAD = 20480
_NB = _NPAD // _B
_IOU_T = 0.5
_PAD_CLS = 100


def _iou_ge(x1c, y1c, x2c, y2c, ac, x1r, y1r, x2r, y2r, ar):
    xx1 = jnp.maximum(x1c, x1r)
    yy1 = jnp.maximum(y1c, y1r)
    xx2 = jnp.minimum(x2c, x2r)
    yy2 = jnp.minimum(y2c, y2r)
    inter = jnp.maximum(xx2 - xx1, 0.0) * jnp.maximum(yy2 - yy1, 0.0)
    iou = inter / (ac + ar - inter)
    return iou >= _IOU_T


def _nms_body(eb_ref, x1_ref, y1_ref, x2_ref, y2_ref, keep_ref,
              area_ref, tri_ref, a_ref, cx1, cy1, cx2, cy2, ca):
    f32 = jnp.float32
    bf16 = jnp.bfloat16
    area_ref[...] = (x2_ref[...] - x1_ref[...]) * (y2_ref[...] - y1_ref[...])
    keep_ref[...] = jnp.ones((_NB, _B), f32)

    ri = jax.lax.broadcasted_iota(jnp.int32, (_B, _B), 0)
    ci = jax.lax.broadcasted_iota(jnp.int32, (_B, _B), 1)
    tri_ref[...] = jnp.where(ci > ri, f32(1), f32(0))

    for b in range(_NB):
        rsl = (pl.ds(b, 1), slice(None))
        x1r = x1_ref[rsl]
        y1r = y1_ref[rsl]
        x2r = x2_ref[rsl]
        y2r = y2_ref[rsl]
        ar = area_ref[rsl]
        cx1[...] = jnp.broadcast_to(x1r, (_B, _B)).T
        cy1[...] = jnp.broadcast_to(y1r, (_B, _B)).T
        cx2[...] = jnp.broadcast_to(x2r, (_B, _B)).T
        cy2[...] = jnp.broadcast_to(y2r, (_B, _B)).T
        ca[...] = jnp.broadcast_to(ar, (_B, _B)).T

        ge = _iou_ge(cx1[...], cy1[...], cx2[...], cy2[...], ca[...],
                     x1r, y1r, x2r, y2r, ar)
        a_ref[...] = (jnp.where(ge, f32(1), f32(0)) * tri_ref[...]).astype(bf16)

        pre = keep_ref[rsl]

        def fix_cond(carry):
            return carry[1]

        def fix_body(carry, pre=pre):
            k, _ = carry
            sup = jnp.dot(k.astype(bf16), a_ref[...],
                          preferred_element_type=f32)
            knew = pre * jnp.where(sup > 0.0, f32(0), f32(1))
            changed = jnp.sum(jnp.abs(knew - k)) > 0.0
            return knew, changed

        kfin, _ = jax.lax.while_loop(fix_cond, fix_body,
                                     (pre, jnp.bool_(True)))
        keep_ref[rsl] = kfin

        if b + 1 >= _NB:
            continue
        kb = kfin.astype(bf16)
        limit = eb_ref[b]

        def cross_cond(j, limit=limit):
            return j <= limit

        def cross_body(j, kb=kb):
            jsl = (pl.ds(j, 1), slice(None))
            x1j = x1_ref[jsl]
            y1j = y1_ref[jsl]
            x2j = x2_ref[jsl]
            y2j = y2_ref[jsl]
            aj = area_ref[jsl]
            gej = _iou_ge(cx1[...], cy1[...], cx2[...], cy2[...], ca[...],
                          x1j, y1j, x2j, y2j, aj)
            ac = jnp.where(gej, f32(1), f32(0)).astype(bf16)
            sup = jnp.dot(kb, ac, preferred_element_type=f32)
            keep_ref[jsl] = keep_ref[jsl] * jnp.where(sup > 0.0, f32(0), f32(1))
            return j + 1

        jax.lax.while_loop(cross_cond, cross_body, jnp.int32(b + 1))


def _run_nms(x1, y1, x2, y2, eb):
    f32 = jnp.float32
    bf16 = jnp.bfloat16
    return pl.pallas_call(
        _nms_body,
        in_specs=[
            pl.BlockSpec(memory_space=pltpu.SMEM),
            pl.BlockSpec(memory_space=pltpu.VMEM),
            pl.BlockSpec(memory_space=pltpu.VMEM),
            pl.BlockSpec(memory_space=pltpu.VMEM),
            pl.BlockSpec(memory_space=pltpu.VMEM),
        ],
        out_shape=jax.ShapeDtypeStruct((_NB, _B), f32),
        scratch_shapes=[
            pltpu.VMEM((_NB, _B), f32),
            pltpu.VMEM((_B, _B), f32),
            pltpu.VMEM((_B, _B), bf16),
            pltpu.VMEM((_B, _B), f32),
            pltpu.VMEM((_B, _B), f32),
            pltpu.VMEM((_B, _B), f32),
            pltpu.VMEM((_B, _B), f32),
            pltpu.VMEM((_B, _B), f32),
        ],
    )(eb, x1, y1, x2, y2)


def kernel(boxes, scores, class_ids):
    f32 = jnp.float32
    idx = jnp.arange(_N, dtype=jnp.int32)
    cls_s, _, pos = jax.lax.sort(
        (class_ids, -scores, idx), num_keys=2, is_stable=True)

    max_coordinate = boxes.max()
    offsets = cls_s.astype(f32) * (max_coordinate + 1.0)
    bb = boxes[pos] + offsets[:, None]

    pad = _NPAD - _N
    zero_pad = jnp.zeros((pad,), f32)
    neg_pad = jnp.full((pad,), -1.0, f32)
    x1 = jnp.concatenate([bb[:, 0], zero_pad]).reshape(_NB, _B)
    y1 = jnp.concatenate([bb[:, 1], zero_pad]).reshape(_NB, _B)
    x2 = jnp.concatenate([bb[:, 2], neg_pad]).reshape(_NB, _B)
    y2 = jnp.concatenate([bb[:, 3], neg_pad]).reshape(_NB, _B)

    cls_p = jnp.concatenate(
        [cls_s, jnp.full((pad,), _PAD_CLS, cls_s.dtype)])
    last_cls = cls_p[_B - 1::_B]
    eb = ((jnp.searchsorted(cls_p, last_cls, side="right") - 1) // _B
          ).astype(jnp.int32)

    keep2 = _run_nms(x1, y1, x2, y2, eb).reshape(-1)[:_N]
    mask = jnp.zeros((_N,), f32).at[pos].set(keep2)
    kept_boxes = boxes * mask[:, None]
    return mask, kept_boxes


# speedup vs baseline: 365.7236x; 1.1680x over previous
import functools

import jax
import jax.numpy as jnp
from jax import lax
from jax.experimental import pallas as pl
from jax.experimental.pallas import tpu as pltpu
from jax.experimental.pallas import tpu_sc as plsc

_N = 20000
_B = 512
_NPAD = 20480
_NB = _NPAD // _B
_IOU_T = 0.5
_PAD_CLS = 100


def _iou_ge(x1c, y1c, x2c, y2c, ac, x1r, y1r, x2r, y2r, ar):
    xx1 = jnp.maximum(x1c, x1r)
    yy1 = jnp.maximum(y1c, y1r)
    xx2 = jnp.minimum(x2c, x2r)
    yy2 = jnp.minimum(y2c, y2r)
    inter = jnp.maximum(xx2 - xx1, 0.0) * jnp.maximum(yy2 - yy1, 0.0)
    iou = inter / (ac + ar - inter)
    return iou >= _IOU_T


def _nms_body(eb_ref, x1_ref, y1_ref, x2_ref, y2_ref, keep_ref,
              area_ref, tri_ref, a_ref, cx1, cy1, cx2, cy2, ca):
    f32 = jnp.float32
    bf16 = jnp.bfloat16
    area_ref[...] = (x2_ref[...] - x1_ref[...]) * (y2_ref[...] - y1_ref[...])
    keep_ref[...] = jnp.ones((_NB, _B), f32)

    ri = jax.lax.broadcasted_iota(jnp.int32, (_B, _B), 0)
    ci = jax.lax.broadcasted_iota(jnp.int32, (_B, _B), 1)
    tri_ref[...] = jnp.where(ci > ri, f32(1), f32(0))

    for b in range(_NB):
        rsl = (pl.ds(b, 1), slice(None))
        x1r = x1_ref[rsl]
        y1r = y1_ref[rsl]
        x2r = x2_ref[rsl]
        y2r = y2_ref[rsl]
        ar = area_ref[rsl]
        cx1[...] = jnp.broadcast_to(x1r, (_B, _B)).T
        cy1[...] = jnp.broadcast_to(y1r, (_B, _B)).T
        cx2[...] = jnp.broadcast_to(x2r, (_B, _B)).T
        cy2[...] = jnp.broadcast_to(y2r, (_B, _B)).T
        ca[...] = jnp.broadcast_to(ar, (_B, _B)).T

        ge = _iou_ge(cx1[...], cy1[...], cx2[...], cy2[...], ca[...],
                     x1r, y1r, x2r, y2r, ar)
        a_ref[...] = (jnp.where(ge, f32(1), f32(0)) * tri_ref[...]).astype(bf16)

        pre = keep_ref[rsl]

        def fix_cond(carry):
            return carry[1]

        def fix_body(carry, pre=pre):
            k, _ = carry
            sup = jnp.dot(k.astype(bf16), a_ref[...],
                          preferred_element_type=f32)
            knew = pre * jnp.where(sup > 0.0, f32(0), f32(1))
            changed = jnp.sum(jnp.abs(knew - k)) > 0.0
            return knew, changed

        kfin, _ = jax.lax.while_loop(fix_cond, fix_body,
                                     (pre, jnp.bool_(True)))
        keep_ref[rsl] = kfin

        if b + 1 >= _NB:
            continue
        kb = kfin.astype(bf16)
        limit = eb_ref[b]

        def cross_cond(j, limit=limit):
            return j <= limit

        def cross_body(j, kb=kb):
            jsl = (pl.ds(j, 1), slice(None))
            x1j = x1_ref[jsl]
            y1j = y1_ref[jsl]
            x2j = x2_ref[jsl]
            y2j = y2_ref[jsl]
            aj = area_ref[jsl]
            gej = _iou_ge(cx1[...], cy1[...], cx2[...], cy2[...], ca[...],
                          x1j, y1j, x2j, y2j, aj)
            ac = jnp.where(gej, f32(1), f32(0)).astype(bf16)
            sup = jnp.dot(kb, ac, preferred_element_type=f32)
            keep_ref[jsl] = keep_ref[jsl] * jnp.where(sup > 0.0, f32(0), f32(1))
            return j + 1

        jax.lax.while_loop(cross_cond, cross_body, jnp.int32(b + 1))


def _run_nms(x1, y1, x2, y2, eb):
    f32 = jnp.float32
    bf16 = jnp.bfloat16
    return pl.pallas_call(
        _nms_body,
        in_specs=[
            pl.BlockSpec(memory_space=pltpu.SMEM),
            pl.BlockSpec(memory_space=pltpu.VMEM),
            pl.BlockSpec(memory_space=pltpu.VMEM),
            pl.BlockSpec(memory_space=pltpu.VMEM),
            pl.BlockSpec(memory_space=pltpu.VMEM),
        ],
        out_shape=jax.ShapeDtypeStruct((_NB, _B), f32),
        scratch_shapes=[
            pltpu.VMEM((_NB, _B), f32),
            pltpu.VMEM((_B, _B), f32),
            pltpu.VMEM((_B, _B), bf16),
            pltpu.VMEM((_B, _B), f32),
            pltpu.VMEM((_B, _B), f32),
            pltpu.VMEM((_B, _B), f32),
            pltpu.VMEM((_B, _B), f32),
            pltpu.VMEM((_B, _B), f32),
        ],
    )(eb, x1, y1, x2, y2)


_NC = 2
_NS = 16
_NW = _NC * _NS
_WROWS = _NPAD // (_NW * 128)


def _sc_scatter_call(keep2d, pos2d):
    mesh = plsc.VectorSubcoreMesh(core_axis_name="c", subcore_axis_name="s",
                                  num_cores=_NC)

    @functools.partial(
        pl.kernel,
        mesh=mesh,
        out_type=jax.ShapeDtypeStruct((_NPAD,), jnp.float32),
        scratch_types=[
            pltpu.VMEM((_WROWS, 128), jnp.int32),
            pltpu.VMEM((_WROWS, 128), jnp.float32),
            pltpu.SemaphoreType.DMA,
        ],
    )
    def body(keep_hbm, pos_hbm, out_hbm, idx_v, val_v, sem):
        wid = lax.axis_index("s") * _NC + lax.axis_index("c")
        pltpu.sync_copy(pos_hbm.at[wid], idx_v)
        pltpu.sync_copy(keep_hbm.at[wid], val_v)
        copies = [
            pltpu.async_copy(val_v.at[j], out_hbm.at[idx_v.at[j]], sem)
            for j in range(_WROWS)
        ]
        for c in copies:
            c.wait()

    return body(keep2d, pos2d)


def kernel(boxes, scores, class_ids):
    f32 = jnp.float32
    idx = jnp.arange(_N, dtype=jnp.int32)
    cls_s, _, pos = jax.lax.sort(
        (class_ids, -scores, idx), num_keys=2, is_stable=True)

    max_coordinate = boxes.max()
    offsets = cls_s.astype(f32) * (max_coordinate + 1.0)
    bb = boxes[pos] + offsets[:, None]

    pad = _NPAD - _N
    zero_pad = jnp.zeros((pad,), f32)
    neg_pad = jnp.full((pad,), -1.0, f32)
    x1 = jnp.concatenate([bb[:, 0], zero_pad]).reshape(_NB, _B)
    y1 = jnp.concatenate([bb[:, 1], zero_pad]).reshape(_NB, _B)
    x2 = jnp.concatenate([bb[:, 2], neg_pad]).reshape(_NB, _B)
    y2 = jnp.concatenate([bb[:, 3], neg_pad]).reshape(_NB, _B)

    cls_p = jnp.concatenate(
        [cls_s, jnp.full((pad,), _PAD_CLS, cls_s.dtype)])
    last_cls = cls_p[_B - 1::_B]
    eb = ((jnp.searchsorted(cls_p, last_cls, side="right") - 1) // _B
          ).astype(jnp.int32)

    keep2d = _run_nms(x1, y1, x2, y2, eb).reshape(_NW, _WROWS, 128)
    pos2d = jnp.concatenate(
        [pos, jnp.arange(_N, _NPAD, dtype=jnp.int32)]).reshape(_NW, _WROWS, 128)
    mask = _sc_scatter_call(keep2d, pos2d)[:_N]
    kept_boxes = boxes * mask[:, None]
    return mask, kept_boxes
